# baseline (speedup 1.0000x reference)
import jax
import jax.numpy as jnp
from jax.experimental import pallas as pl
from jax.experimental.pallas import tpu as pltpu

_DN_T = (((1,), (1,)), ((), ()))

_CH = 512


def _pipeline(x_hbm, w1_ref, b1_ref, w2_ref, b2_ref, w3_ref, b3_ref,
              wc_ref, f_hbm, s_hbm, xv, fv, sv, sem_in, sem_f, sem_s):
    n, d = x_hbm.shape
    n_chunks = n // _CH

    def _in_copy(ci):
        return pltpu.make_async_copy(
            x_hbm.at[pl.ds(ci * _CH, _CH), :], xv.at[ci], sem_in.at[ci])

    def _f_copy(ci):
        return pltpu.make_async_copy(
            fv.at[ci], f_hbm.at[pl.ds(ci * _CH, _CH), :], sem_f.at[ci])

    def _s_copy(ci):
        return pltpu.make_async_copy(
            sv.at[ci], s_hbm.at[pl.ds(ci * _CH, _CH), :], sem_s.at[ci])

    for ci in range(n_chunks):
        _in_copy(ci).start()

    def step(ci, carry):
        _in_copy(ci).wait()
        h = jax.lax.dot_general(xv[ci], w1_ref[...], _DN_T,
                                preferred_element_type=jnp.float32)
        h = jnp.maximum(h + b1_ref[...], 0.0)
        h = jax.lax.dot_general(h, w2_ref[...], _DN_T,
                                preferred_element_type=jnp.float32)
        h = jnp.maximum(h + b2_ref[...], 0.0)
        f = jax.lax.dot_general(h, w3_ref[...], _DN_T,
                                preferred_element_type=jnp.float32)
        f = jnp.maximum(f + b3_ref[...], 0.0)
        s = jax.lax.dot_general(f, wc_ref[...], _DN_T,
                                preferred_element_type=jnp.float32)
        fv[ci] = f
        sv[ci] = s
        _f_copy(ci).start()
        _s_copy(ci).start(priority=1)
        return carry

    jax.lax.fori_loop(0, n_chunks, step, 0, unroll=True)

    for ci in range(n_chunks):
        _f_copy(ci).wait()
        _s_copy(ci).wait()


def kernel(x, W1, b1, W2, b2, W3, b3, Wc):
    N, D = x.shape
    H1 = W1.shape[0]
    H2 = W2.shape[0]
    H3 = W3.shape[0]
    C = Wc.shape[0]
    n_chunks = N // _CH

    hbm = pl.BlockSpec(memory_space=pltpu.MemorySpace.HBM)
    vmem = pl.BlockSpec(memory_space=pltpu.MemorySpace.VMEM)

    features, scores = pl.pallas_call(
        _pipeline,
        in_specs=[hbm, vmem, vmem, vmem, vmem, vmem, vmem, vmem],
        out_specs=[hbm, hbm],
        out_shape=[
            jax.ShapeDtypeStruct((N, H3), jnp.float32),
            jax.ShapeDtypeStruct((N, C), jnp.float32),
        ],
        scratch_shapes=[
            pltpu.VMEM((n_chunks, _CH, D), jnp.float32),
            pltpu.VMEM((n_chunks, _CH, H3), jnp.float32),
            pltpu.VMEM((n_chunks, _CH, C), jnp.float32),
            pltpu.SemaphoreType.DMA((n_chunks,)),
            pltpu.SemaphoreType.DMA((n_chunks,)),
            pltpu.SemaphoreType.DMA((n_chunks,)),
        ],
    )(x, W1, b1, W2, b2, W3, b3, Wc)
    return (features, scores)


# speedup vs baseline: 1.5150x; 1.5150x over previous
import jax
import jax.numpy as jnp
from jax.experimental import pallas as pl
from jax.experimental.pallas import tpu as pltpu

_DN_T = (((1,), (1,)), ((), ()))

_CH = 4096


def _pipeline(x_hbm, w1_ref, b1_ref, w2_ref, b2_ref, w3_ref, b3_ref,
              wc_ref, f_hbm, s_hbm, xv, fv, sv, sem_in, sem_f, sem_s):
    n, d = x_hbm.shape
    n_chunks = n // _CH

    def _in_copy(ci):
        return pltpu.make_async_copy(
            x_hbm.at[pl.ds(ci * _CH, _CH), :], xv.at[ci], sem_in.at[ci])

    def _f_copy(ci):
        return pltpu.make_async_copy(
            fv.at[ci], f_hbm.at[pl.ds(ci * _CH, _CH), :], sem_f.at[ci])

    def _s_copy(ci):
        return pltpu.make_async_copy(
            sv.at[ci], s_hbm.at[pl.ds(ci * _CH, _CH), :], sem_s.at[ci])

    for ci in range(n_chunks):
        _in_copy(ci).start()

    def step(ci, carry):
        _in_copy(ci).wait()
        h = jax.lax.dot_general(xv[ci], w1_ref[...], _DN_T,
                                preferred_element_type=jnp.float32)
        h = jnp.maximum(h + b1_ref[...], 0.0)
        h = jax.lax.dot_general(h, w2_ref[...], _DN_T,
                                preferred_element_type=jnp.float32)
        h = jnp.maximum(h + b2_ref[...], 0.0)
        f = jax.lax.dot_general(h, w3_ref[...], _DN_T,
                                preferred_element_type=jnp.float32)
        f = jnp.maximum(f + b3_ref[...], 0.0)
        s = jax.lax.dot_general(f, wc_ref[...], _DN_T,
                                preferred_element_type=jnp.float32)
        fv[ci] = f
        sv[ci] = s
        _f_copy(ci).start()
        _s_copy(ci).start(priority=1)
        return carry

    jax.lax.fori_loop(0, n_chunks, step, 0, unroll=True)

    for ci in range(n_chunks):
        _f_copy(ci).wait()
        _s_copy(ci).wait()


def kernel(x, W1, b1, W2, b2, W3, b3, Wc):
    N, D = x.shape
    H1 = W1.shape[0]
    H2 = W2.shape[0]
    H3 = W3.shape[0]
    C = Wc.shape[0]
    n_chunks = N // _CH

    hbm = pl.BlockSpec(memory_space=pltpu.MemorySpace.HBM)
    vmem = pl.BlockSpec(memory_space=pltpu.MemorySpace.VMEM)

    features, scores = pl.pallas_call(
        _pipeline,
        in_specs=[hbm, vmem, vmem, vmem, vmem, vmem, vmem, vmem],
        out_specs=[hbm, hbm],
        out_shape=[
            jax.ShapeDtypeStruct((N, H3), jnp.float32),
            jax.ShapeDtypeStruct((N, C), jnp.float32),
        ],
        scratch_shapes=[
            pltpu.VMEM((n_chunks, _CH, D), jnp.float32),
            pltpu.VMEM((n_chunks, _CH, H3), jnp.float32),
            pltpu.VMEM((n_chunks, _CH, C), jnp.float32),
            pltpu.SemaphoreType.DMA((n_chunks,)),
            pltpu.SemaphoreType.DMA((n_chunks,)),
            pltpu.SemaphoreType.DMA((n_chunks,)),
        ],
    )(x, W1, b1, W2, b2, W3, b3, Wc)
    return (features, scores)
